# baseline (speedup 1.0000x reference)
import functools

import jax
import jax.numpy as jnp
from jax.experimental import pallas as pl
from jax.experimental.pallas import tpu as pltpu

N_EXPERTS = 64
TOP_K = 8


def _gate_kernel(x_ref, wt_ref, idx_ref, w_ref):
    x = x_ref[...]
    logits = jnp.dot(x, wt_ref[...], preferred_element_type=jnp.float32)
    lt = logits.T
    m = jnp.max(lt, axis=0, keepdims=True)
    ex = jnp.exp(lt - m)
    e = ex / jnp.sum(ex, axis=0, keepdims=True)

    ids_f = jax.lax.broadcasted_iota(jnp.int32, e.shape, 0).astype(
        jnp.float32
    )
    vals = []
    idxs = []
    work = e
    for _ in range(TOP_K):
        v = jnp.max(work, axis=0, keepdims=True)
        hit = work == v
        i = jnp.min(
            jnp.where(hit, ids_f, jnp.float32(N_EXPERTS)),
            axis=0,
            keepdims=True,
        )
        vals.append(v)
        idxs.append(i)
        work = jnp.where(hit & (ids_f == i), -jnp.inf, work)

    topv = jnp.concatenate(vals, axis=0)
    topi = jnp.concatenate(idxs, axis=0).astype(jnp.int32)
    denom = jnp.sum(topv, axis=0, keepdims=True) + 1e-20
    topw = topv / denom
    idx_ref[...] = topi.T
    w_ref[...] = topw.T


@functools.partial(jax.jit, static_argnames=())
def kernel(hidden_states, weight):
    bsz, seq, h = hidden_states.shape
    t = bsz * seq
    x = hidden_states.reshape(t, h)
    wt = weight.T

    bt = 4096
    grid = (t // bt,)

    idx, w = pl.pallas_call(
        _gate_kernel,
        grid=grid,
        in_specs=[
            pl.BlockSpec((bt, h), lambda i: (i, 0)),
            pl.BlockSpec((h, N_EXPERTS), lambda i: (0, 0)),
        ],
        out_specs=[
            pl.BlockSpec((bt, TOP_K), lambda i: (i, 0)),
            pl.BlockSpec((bt, TOP_K), lambda i: (i, 0)),
        ],
        out_shape=[
            jax.ShapeDtypeStruct((t, TOP_K), jnp.int32),
            jax.ShapeDtypeStruct((t, TOP_K), jnp.float32),
        ],
        compiler_params=pltpu.CompilerParams(
            dimension_semantics=("parallel",),
        ),
    )(x, wt)

    return (idx.reshape(bsz, seq, TOP_K), w.reshape(bsz, seq, TOP_K))


# speedup vs baseline: 1.0587x; 1.0587x over previous
import functools

import jax
import jax.numpy as jnp
from jax.experimental import pallas as pl
from jax.experimental.pallas import tpu as pltpu

N_EXPERTS = 64
TOP_K = 8


def _gate_kernel(x_ref, wt_ref, idx_ref, w_ref):
    x = x_ref[...]
    logits = jnp.dot(x, wt_ref[...], preferred_element_type=jnp.float32)
    lt = logits.T
    m = jnp.max(lt, axis=0, keepdims=True)
    ex = jnp.exp(lt - m)
    e = ex / (jnp.sum(ex, axis=0, keepdims=True) * jnp.float32(2.0**96))

    rev_ids = (N_EXPERTS - 1) - jax.lax.broadcasted_iota(
        jnp.int32, e.shape, 0
    )
    bits = jax.lax.bitcast_convert_type(e, jnp.int32)
    keys = jax.lax.bitcast_convert_type(
        ((bits << 3) & ~(N_EXPERTS - 1)) | rev_ids, jnp.float32
    )

    kmaxs = []
    for _ in range(TOP_K):
        kmax = jnp.max(keys, axis=0, keepdims=True)
        kmaxs.append(kmax)
        keys = jnp.where(keys == kmax, -jnp.inf, keys)

    kcat = jax.lax.bitcast_convert_type(
        jnp.concatenate(kmaxs, axis=0), jnp.int32
    )
    topi = (N_EXPERTS - 1) - (kcat & (N_EXPERTS - 1))
    topv = jax.lax.bitcast_convert_type(
        (kcat & ~(N_EXPERTS - 1)) >> 3, jnp.float32
    ) * jnp.float32(2.0**96)
    denom = jnp.sum(topv, axis=0, keepdims=True) + 1e-20
    topw = topv / denom
    idx_ref[...] = topi.T
    w_ref[...] = topw.T


@functools.partial(jax.jit, static_argnames=())
def kernel(hidden_states, weight):
    bsz, seq, h = hidden_states.shape
    t = bsz * seq
    x = hidden_states.reshape(t, h)
    wt = weight.T

    bt = 4096
    grid = (t // bt,)

    idx, w = pl.pallas_call(
        _gate_kernel,
        grid=grid,
        in_specs=[
            pl.BlockSpec((bt, h), lambda i: (i, 0)),
            pl.BlockSpec((h, N_EXPERTS), lambda i: (0, 0)),
        ],
        out_specs=[
            pl.BlockSpec((bt, TOP_K), lambda i: (i, 0)),
            pl.BlockSpec((bt, TOP_K), lambda i: (i, 0)),
        ],
        out_shape=[
            jax.ShapeDtypeStruct((t, TOP_K), jnp.int32),
            jax.ShapeDtypeStruct((t, TOP_K), jnp.float32),
        ],
        compiler_params=pltpu.CompilerParams(
            dimension_semantics=("parallel",),
        ),
    )(x, wt)

    return (idx.reshape(bsz, seq, TOP_K), w.reshape(bsz, seq, TOP_K))
